# Initial kernel scaffold; baseline (speedup 1.0000x reference)
import jax
import jax.numpy as jnp
from jax.experimental import pallas as pl


def kernel(x, edge_index, W1, b1, W2, b2, W3, b3, W4, b4, W5, b5):
    raise NotImplementedError("write your pallas kernel here")


# speedup vs baseline: 6.4330x; 6.4330x over previous
import functools

import jax
import jax.numpy as jnp
from jax import lax
from jax.experimental import pallas as pl
from jax.experimental.pallas import tpu as pltpu
from jax.experimental.pallas import tpu_sc as plsc

N = 10000
D = 128
NCLS = 40
NCLS_PAD = 128
K = 128
NTILES = 16
NCORES = 2
NW = NCORES * NTILES
N_PAD = 10112
RPT = N_PAD // NTILES
BR = 1000

_ZBLK = []
_off = 0
while _off < RPT:
    _ZBLK.append((_off, min(K, RPT - _off)))
    _off += _ZBLK[-1][1]
_ZBLK = tuple(_ZBLK)


def _make_deg(C):
    mesh = plsc.VectorSubcoreMesh(core_axis_name="c", subcore_axis_name="s", num_cores=NCORES, num_subcores=NTILES)

    @functools.partial(
        pl.kernel, mesh=mesh,
        out_type=jax.ShapeDtypeStruct((NCORES, N_PAD, 16), jnp.float32),
        scratch_types=[
            pltpu.VMEM((C, K), jnp.int32),
            pltpu.VMEM((K, 16), jnp.float32),
            pltpu.VMEM_SHARED((N_PAD, 16), jnp.float32),
        ])
    def deg(dst_hbm, out_hbm, dst_v, ones_v, acc):
        c = lax.axis_index("c")
        s = lax.axis_index("s")
        w = c * NTILES + s

        def zrow(i, carry):
            ones_v[i, pl.ds(0, 16)] = jnp.zeros((16,), jnp.float32)
            return carry
        lax.fori_loop(0, K, zrow, 0)
        base = s * RPT
        for off, sz in _ZBLK:
            pltpu.sync_copy(ones_v.at[pl.ds(0, sz)], acc.at[pl.ds(base + off, sz)])

        def orow(i, carry):
            ones_v[i, pl.ds(0, 16)] = jnp.full((16,), 1.0, jnp.float32)
            return carry
        lax.fori_loop(0, K, orow, 0)
        pltpu.sync_copy(dst_hbm.at[pl.ds(w * C, C)], dst_v)
        plsc.subcore_barrier()

        def body(j, carry):
            pltpu.sync_copy(ones_v, acc.at[dst_v.at[j]], add=True)
            return carry
        lax.fori_loop(0, C, body, 0)
        plsc.subcore_barrier()
        for off, sz in _ZBLK:
            pltpu.sync_copy(acc.at[pl.ds(base + off, sz)],
                            out_hbm.at[c, pl.ds(base + off, sz)])

    return deg


def _make_seg(Df, C):
    mesh = plsc.VectorSubcoreMesh(core_axis_name="c", subcore_axis_name="s", num_cores=NCORES, num_subcores=NTILES)

    @functools.partial(
        pl.kernel, mesh=mesh,
        out_type=jax.ShapeDtypeStruct((NCORES, N_PAD, Df), jnp.float32),
        scratch_types=[
            pltpu.VMEM((C, K), jnp.int32),
            pltpu.VMEM((C, K), jnp.int32),
            pltpu.VMEM((K, Df), jnp.float32),
            pltpu.VMEM_SHARED((N_PAD, Df), jnp.float32),
            pltpu.SemaphoreType.DMA,
        ])
    def seg(g_hbm, src_hbm, dst_hbm, out_hbm, src_v, dst_v, rows_v, acc, sem):
        c = lax.axis_index("c")
        s = lax.axis_index("s")
        w = c * NTILES + s
        nz = Df // 16

        def zrow(i, carry):
            for q in range(nz):
                rows_v[i, pl.ds(q * 16, 16)] = jnp.zeros((16,), jnp.float32)
            return carry
        lax.fori_loop(0, K, zrow, 0)
        base = s * RPT
        for off, sz in _ZBLK:
            pltpu.sync_copy(rows_v.at[pl.ds(0, sz)], acc.at[pl.ds(base + off, sz)])
        pltpu.sync_copy(src_hbm.at[pl.ds(w * C, C)], src_v)
        pltpu.sync_copy(dst_hbm.at[pl.ds(w * C, C)], dst_v)
        plsc.subcore_barrier()

        def body(j, carry):
            pltpu.async_copy(g_hbm.at[src_v.at[j]], rows_v, sem).wait()
            pltpu.sync_copy(rows_v, acc.at[dst_v.at[j]], add=True)
            return carry
        lax.fori_loop(0, C, body, 0)
        plsc.subcore_barrier()
        for off, sz in _ZBLK:
            pltpu.sync_copy(acc.at[pl.ds(base + off, sz)],
                            out_hbm.at[c, pl.ds(base + off, sz)])

    return seg


def _dinv_of(deg_ref):
    return lax.rsqrt(deg_ref[0, :, 0:1] + deg_ref[1, :, 0:1] + 1.0)


def _mm1_body(x_ref, w_ref, deg_ref, o_ref):
    o_ref[:] = jnp.dot(x_ref[:], w_ref[:],
                       preferred_element_type=jnp.float32,
                       precision=lax.Precision.HIGHEST) * _dinv_of(deg_ref)


def _mid_body(s_ref, g_ref, deg_ref, b_ref, w_ref, o_ref):
    dinv = _dinv_of(deg_ref)
    h = (s_ref[0] + s_ref[1] + g_ref[:]) * dinv + b_ref[:]
    h = jnp.maximum(h, 0.0)
    o_ref[:] = jnp.dot(h, w_ref[:],
                       preferred_element_type=jnp.float32,
                       precision=lax.Precision.HIGHEST) * dinv


def _fin_body(s_ref, g_ref, deg_ref, b_ref, o_ref):
    dinv = _dinv_of(deg_ref)
    h = (s_ref[0] + s_ref[1] + g_ref[:]) * dinv + b_ref[:]
    z = h[:, :NCLS]
    m = jnp.max(z, axis=1, keepdims=True)
    lse = jnp.log(jnp.sum(jnp.exp(z - m), axis=1, keepdims=True)) + m
    o_ref[:] = z - lse


def _mm1(x, w1, deg):
    return pl.pallas_call(
        _mm1_body,
        grid=(N // BR,),
        in_specs=[
            pl.BlockSpec((BR, D), lambda i: (i, 0)),
            pl.BlockSpec((D, D), lambda i: (0, 0)),
            pl.BlockSpec((NCORES, BR, 16), lambda i: (0, i, 0)),
        ],
        out_specs=pl.BlockSpec((BR, D), lambda i: (i, 0)),
        out_shape=jax.ShapeDtypeStruct((N, D), jnp.float32),
    )(x, w1, deg)


def _mid(s, g, deg, b, w):
    din, dout = w.shape
    return pl.pallas_call(
        _mid_body,
        grid=(N // BR,),
        in_specs=[
            pl.BlockSpec((NCORES, BR, din), lambda i: (0, i, 0)),
            pl.BlockSpec((BR, din), lambda i: (i, 0)),
            pl.BlockSpec((NCORES, BR, 16), lambda i: (0, i, 0)),
            pl.BlockSpec((1, din), lambda i: (0, 0)),
            pl.BlockSpec((din, dout), lambda i: (0, 0)),
        ],
        out_specs=pl.BlockSpec((BR, dout), lambda i: (i, 0)),
        out_shape=jax.ShapeDtypeStruct((N, dout), jnp.float32),
    )(s, g, deg, b.reshape(1, din), w)


def _fin(s, g, deg, b):
    return pl.pallas_call(
        _fin_body,
        grid=(N // BR,),
        in_specs=[
            pl.BlockSpec((NCORES, BR, NCLS_PAD), lambda i: (0, i, 0)),
            pl.BlockSpec((BR, NCLS_PAD), lambda i: (i, 0)),
            pl.BlockSpec((NCORES, BR, 16), lambda i: (0, i, 0)),
            pl.BlockSpec((1, NCLS_PAD), lambda i: (0, 0)),
        ],
        out_specs=pl.BlockSpec((BR, NCLS), lambda i: (i, 0)),
        out_shape=jax.ShapeDtypeStruct((N, NCLS), jnp.float32),
    )(s, g, deg, b.reshape(1, NCLS_PAD))


def kernel(x, edge_index, W1, b1, W2, b2, W3, b3, W4, b4, W5, b5):
    src = edge_index[0].astype(jnp.int32)
    dst = edge_index[1].astype(jnp.int32)
    e = src.shape[0]
    ch = -(-e // K)
    c_per_w = ((ch + NW - 1) // NW + 7) // 8 * 8
    ch_pad = c_per_w * NW
    ep = ch_pad * K
    src_p = jnp.concatenate(
        [src, jnp.zeros((ep - e,), jnp.int32)]).reshape(ch_pad, K)
    dst_p = jnp.concatenate(
        [dst, jnp.full((ep - e,), N, jnp.int32)]).reshape(ch_pad, K)

    w5p = jnp.pad(W5, ((0, 0), (0, NCLS_PAD - NCLS)))
    b5p = jnp.pad(b5, (0, NCLS_PAD - NCLS))

    deg = _make_deg(c_per_w)(dst_p)
    seg128 = _make_seg(D, c_per_w)

    g = _mm1(x, W1, deg)
    for b, w in ((b1, W2), (b2, W3), (b3, W4), (b4, w5p)):
        s = seg128(g, src_p, dst_p)
        g = _mid(s, g, deg, b, w)
    s = seg128(g, src_p, dst_p)
    return _fin(s, g, deg, b5p)


# speedup vs baseline: 23.7622x; 3.6938x over previous
import functools

import jax
import jax.numpy as jnp
from jax import lax
from jax.experimental import pallas as pl
from jax.experimental.pallas import tpu as pltpu
from jax.experimental.pallas import tpu_sc as plsc

N = 10000
D = 128
NCLS = 40
NCLS_PAD = 128
K = 128
NTILES = 16
NCORES = 2
NW = NCORES * NTILES
N_PAD = 10112
RPT = N_PAD // NTILES
BR = 1000

_ZBLK = []
_off = 0
while _off < RPT:
    _ZBLK.append((_off, min(K, RPT - _off)))
    _off += _ZBLK[-1][1]
_ZBLK = tuple(_ZBLK)


def _n_real(w, C, ch_proc):
    return jnp.clip(ch_proc - w * C, 0, C)


def _make_deg(C, ch_proc):
    mesh = plsc.VectorSubcoreMesh(core_axis_name="c", subcore_axis_name="s", num_cores=NCORES, num_subcores=NTILES)

    @functools.partial(
        pl.kernel, mesh=mesh,
        out_type=jax.ShapeDtypeStruct((NCORES, N_PAD, 16), jnp.float32),
        scratch_types=[
            pltpu.VMEM((C, K), jnp.int32),
            pltpu.VMEM((K, 16), jnp.float32),
            pltpu.VMEM_SHARED((N_PAD, 16), jnp.float32),
        ])
    def deg(dst_hbm, out_hbm, dst_v, ones_v, acc):
        c = lax.axis_index("c")
        s = lax.axis_index("s")
        w = c * NTILES + s
        nreal = _n_real(w, C, ch_proc)

        def zrow(i, carry):
            ones_v[i, pl.ds(0, 16)] = jnp.zeros((16,), jnp.float32)
            return carry
        lax.fori_loop(0, K, zrow, 0)
        base = s * RPT
        for off, sz in _ZBLK:
            pltpu.sync_copy(ones_v.at[pl.ds(0, sz)], acc.at[pl.ds(base + off, sz)])

        def orow(i, carry):
            ones_v[i, pl.ds(0, 16)] = jnp.full((16,), 1.0, jnp.float32)
            return carry
        lax.fori_loop(0, K, orow, 0)
        pltpu.sync_copy(dst_hbm.at[pl.ds(w * C, C)], dst_v)
        plsc.subcore_barrier()

        def body(j, carry):
            pltpu.sync_copy(ones_v, acc.at[dst_v.at[j]], add=True)
            return carry
        lax.fori_loop(0, nreal, body, 0)
        plsc.subcore_barrier()
        for off, sz in _ZBLK:
            pltpu.sync_copy(acc.at[pl.ds(base + off, sz)],
                            out_hbm.at[c, pl.ds(base + off, sz)])

    return deg


def _make_seg(Df, C, ch_proc):
    NBUF = 2
    NHALF = 2
    C2 = C // NHALF
    mesh = plsc.VectorSubcoreMesh(core_axis_name="c", subcore_axis_name="s", num_cores=NCORES, num_subcores=NTILES)

    @functools.partial(
        pl.kernel, mesh=mesh,
        out_type=jax.ShapeDtypeStruct((NCORES, N_PAD, Df), jnp.float32),
        scratch_types=[
            pltpu.VMEM((C2, K), jnp.int32),
            pltpu.VMEM((C2, K), jnp.int32),
        ] + [pltpu.VMEM((K, Df), jnp.float32)] * NBUF
          + [pltpu.VMEM_SHARED((N_PAD, Df), jnp.float32)]
          + [pltpu.SemaphoreType.DMA] * (2 * NBUF),
    )
    def seg(g_hbm, src_hbm, dst_hbm, out_hbm, src_v, dst_v, *bufs_sems):
        rows = bufs_sems[:NBUF]
        acc = bufs_sems[NBUF]
        gsem = bufs_sems[NBUF + 1:2 * NBUF + 1]
        ssem = bufs_sems[2 * NBUF + 1:3 * NBUF + 1]
        c = lax.axis_index("c")
        s = lax.axis_index("s")
        w = c * NTILES + s
        nz = Df // 16

        def zrow(i, carry):
            for q in range(nz):
                rows[0][i, pl.ds(q * 16, 16)] = jnp.zeros((16,), jnp.float32)
            return carry
        lax.fori_loop(0, K, zrow, 0)
        base = s * RPT
        for off, sz in _ZBLK:
            pltpu.sync_copy(rows[0].at[pl.ds(0, sz)],
                            acc.at[pl.ds(base + off, sz)])
        plsc.subcore_barrier()

        for half in range(NHALF):
            cb = w * C + half * C2
            nreal = jnp.clip(ch_proc - cb, 0, C2)
            pltpu.sync_copy(src_hbm.at[pl.ds(cb, C2)], src_v)
            pltpu.sync_copy(dst_hbm.at[pl.ds(cb, C2)], dst_v)

            for b in range(NBUF):
                @pl.when(b < nreal)
                def _(b=b):
                    pltpu.async_copy(g_hbm.at[src_v.at[b]], rows[b], gsem[b])

            def group(gi, carry):
                for b in range(NBUF):
                    j = gi * NBUF + b
                    pltpu.make_async_copy(
                        g_hbm.at[src_v.at[j]], rows[b], gsem[b]).wait()
                    pltpu.async_copy(
                        rows[b], acc.at[dst_v.at[j]], ssem[b], add=True)
                    jn = j + NBUF

                    @pl.when(jn < nreal)
                    def _(b=b, j=j, jn=jn):
                        pltpu.make_async_copy(
                            rows[b], acc.at[dst_v.at[j]], ssem[b]).wait()
                        pltpu.async_copy(
                            g_hbm.at[src_v.at[jn]], rows[b], gsem[b])
                return carry
            lax.fori_loop(0, nreal // NBUF, group, 0)
            for b in range(NBUF):
                @pl.when(b < nreal)
                def _(b=b):
                    pltpu.make_async_copy(
                        rows[b], acc.at[dst_v.at[0]], ssem[b]).wait()
        plsc.subcore_barrier()
        for off, sz in _ZBLK:
            pltpu.sync_copy(acc.at[pl.ds(base + off, sz)],
                            out_hbm.at[c, pl.ds(base + off, sz)])

    return seg


def _dinv_of(deg_ref):
    return lax.rsqrt(deg_ref[0, :, 0:1] + deg_ref[1, :, 0:1] + 1.0)


def _mm1_body(x_ref, w_ref, deg_ref, o_ref):
    o_ref[:] = jnp.dot(x_ref[:], w_ref[:],
                       preferred_element_type=jnp.float32,
                       precision=lax.Precision.HIGHEST) * _dinv_of(deg_ref)


def _mid_body(s_ref, g_ref, deg_ref, b_ref, w_ref, o_ref):
    dinv = _dinv_of(deg_ref)
    h = (s_ref[0] + s_ref[1] + g_ref[:]) * dinv + b_ref[:]
    h = jnp.maximum(h, 0.0)
    o_ref[:] = jnp.dot(h, w_ref[:],
                       preferred_element_type=jnp.float32,
                       precision=lax.Precision.HIGHEST) * dinv


def _fin_body(s_ref, g_ref, deg_ref, b_ref, o_ref):
    dinv = _dinv_of(deg_ref)
    h = (s_ref[0] + s_ref[1] + g_ref[:]) * dinv + b_ref[:]
    z = h[:, :NCLS]
    m = jnp.max(z, axis=1, keepdims=True)
    lse = jnp.log(jnp.sum(jnp.exp(z - m), axis=1, keepdims=True)) + m
    o_ref[:] = z - lse


def _mm1(x, w1, deg):
    return pl.pallas_call(
        _mm1_body,
        grid=(N // BR,),
        in_specs=[
            pl.BlockSpec((BR, D), lambda i: (i, 0)),
            pl.BlockSpec((D, D), lambda i: (0, 0)),
            pl.BlockSpec((NCORES, BR, 16), lambda i: (0, i, 0)),
        ],
        out_specs=pl.BlockSpec((BR, D), lambda i: (i, 0)),
        out_shape=jax.ShapeDtypeStruct((N, D), jnp.float32),
    )(x, w1, deg)


def _mid(s, g, deg, b, w):
    din, dout = w.shape
    return pl.pallas_call(
        _mid_body,
        grid=(N // BR,),
        in_specs=[
            pl.BlockSpec((NCORES, BR, din), lambda i: (0, i, 0)),
            pl.BlockSpec((BR, din), lambda i: (i, 0)),
            pl.BlockSpec((NCORES, BR, 16), lambda i: (0, i, 0)),
            pl.BlockSpec((1, din), lambda i: (0, 0)),
            pl.BlockSpec((din, dout), lambda i: (0, 0)),
        ],
        out_specs=pl.BlockSpec((BR, dout), lambda i: (i, 0)),
        out_shape=jax.ShapeDtypeStruct((N, dout), jnp.float32),
    )(s, g, deg, b.reshape(1, din), w)


def _fin(s, g, deg, b):
    return pl.pallas_call(
        _fin_body,
        grid=(N // BR,),
        in_specs=[
            pl.BlockSpec((NCORES, BR, NCLS_PAD), lambda i: (0, i, 0)),
            pl.BlockSpec((BR, NCLS_PAD), lambda i: (i, 0)),
            pl.BlockSpec((NCORES, BR, 16), lambda i: (0, i, 0)),
            pl.BlockSpec((1, NCLS_PAD), lambda i: (0, 0)),
        ],
        out_specs=pl.BlockSpec((BR, NCLS), lambda i: (i, 0)),
        out_shape=jax.ShapeDtypeStruct((N, NCLS), jnp.float32),
    )(s, g, deg, b.reshape(1, NCLS_PAD))


def kernel(x, edge_index, W1, b1, W2, b2, W3, b3, W4, b4, W5, b5):
    src = edge_index[0].astype(jnp.int32)
    dst = edge_index[1].astype(jnp.int32)
    e = src.shape[0]
    ch = -(-e // K)
    c_per_w = ((ch + NW - 1) // NW + 7) // 8 * 8
    ch_pad = c_per_w * NW
    ep = ch_pad * K
    ch_proc = -(-ch // 4) * 4
    pad_dst = N + 1 + (jnp.arange(ep - e, dtype=jnp.int32) % (N_PAD - N - 1))
    src_p = jnp.concatenate(
        [src, jnp.zeros((ep - e,), jnp.int32)]).reshape(ch_pad, K)
    dst_p = jnp.concatenate([dst, pad_dst]).reshape(ch_pad, K)

    w5p = jnp.pad(W5, ((0, 0), (0, NCLS_PAD - NCLS)))
    b5p = jnp.pad(b5, (0, NCLS_PAD - NCLS))

    deg = _make_deg(c_per_w, ch_proc)(dst_p)
    seg128 = _make_seg(D, c_per_w, ch_proc)

    g = _mm1(x, W1, deg)
    for b, w in ((b1, W2), (b2, W3), (b3, W4), (b4, w5p)):
        s = seg128(g, src_p, dst_p)
        g = _mid(s, g, deg, b, w)
    s = seg128(g, src_p, dst_p)
    return _fin(s, g, deg, b5p)


# speedup vs baseline: 24.6387x; 1.0369x over previous
import functools

import jax
import jax.numpy as jnp
from jax import lax
from jax.experimental import pallas as pl
from jax.experimental.pallas import tpu as pltpu
from jax.experimental.pallas import tpu_sc as plsc

N = 10000
D = 128
NCLS = 40
NCLS_PAD = 128
K = 128
NTILES = 16
NCORES = 2
NW = NCORES * NTILES
N_PAD = 10112
RPT = N_PAD // NTILES
BR = 2000

_ZBLK = []
_off = 0
while _off < RPT:
    _ZBLK.append((_off, min(K, RPT - _off)))
    _off += _ZBLK[-1][1]
_ZBLK = tuple(_ZBLK)


def _n_real(w, C, ch_proc):
    return jnp.clip(ch_proc - w * C, 0, C)


def _make_deg(C, ch_proc):
    mesh = plsc.VectorSubcoreMesh(core_axis_name="c", subcore_axis_name="s", num_cores=NCORES, num_subcores=NTILES)

    @functools.partial(
        pl.kernel, mesh=mesh,
        out_type=jax.ShapeDtypeStruct((NCORES, N_PAD, 16), jnp.float32),
        scratch_types=[
            pltpu.VMEM((C, K), jnp.int32),
            pltpu.VMEM((K, 16), jnp.float32),
            pltpu.VMEM_SHARED((N_PAD, 16), jnp.float32),
            pltpu.SemaphoreType.DMA,
        ])
    def deg(dst_hbm, out_hbm, dst_v, ones_v, acc, dsem):
        c = lax.axis_index("c")
        s = lax.axis_index("s")
        w = c * NTILES + s
        nreal = _n_real(w, C, ch_proc)

        def zrow(i, carry):
            ones_v[i, pl.ds(0, 16)] = jnp.zeros((16,), jnp.float32)
            return carry
        lax.fori_loop(0, K, zrow, 0)
        base = s * RPT
        for off, sz in _ZBLK:
            pltpu.sync_copy(ones_v.at[pl.ds(0, sz)], acc.at[pl.ds(base + off, sz)])

        def orow(i, carry):
            ones_v[i, pl.ds(0, 16)] = jnp.full((16,), 1.0, jnp.float32)
            return carry
        lax.fori_loop(0, K, orow, 0)
        pltpu.sync_copy(dst_hbm.at[pl.ds(w * C, C)], dst_v)
        plsc.subcore_barrier()

        def body(j8, carry):
            for q in range(8):
                @pl.when(j8 * 8 + q < nreal)
                def _(q=q):
                    pltpu.async_copy(ones_v, acc.at[dst_v.at[j8 * 8 + q]],
                                     dsem, add=True)
            for q in range(8):
                @pl.when(j8 * 8 + q < nreal)
                def _(q=q):
                    pltpu.make_async_copy(ones_v, acc.at[dst_v.at[0]],
                                          dsem).wait()
            return carry
        lax.fori_loop(0, (nreal + 7) // 8, body, 0)
        plsc.subcore_barrier()
        for off, sz in _ZBLK:
            pltpu.sync_copy(acc.at[pl.ds(base + off, sz)],
                            out_hbm.at[c, pl.ds(base + off, sz)])

    return deg


def _make_seg(Df, C, ch_proc):
    NBUF = 2
    NHALF = 2
    C2 = C // NHALF
    mesh = plsc.VectorSubcoreMesh(core_axis_name="c", subcore_axis_name="s", num_cores=NCORES, num_subcores=NTILES)

    @functools.partial(
        pl.kernel, mesh=mesh,
        out_type=jax.ShapeDtypeStruct((NCORES, N_PAD, Df), jnp.float32),
        scratch_types=[
            pltpu.VMEM((C2, K), jnp.int32),
            pltpu.VMEM((C2, K), jnp.int32),
        ] + [pltpu.VMEM((K, Df), jnp.float32)] * NBUF
          + [pltpu.VMEM_SHARED((N_PAD, Df), jnp.float32)]
          + [pltpu.SemaphoreType.DMA] * (2 * NBUF),
    )
    def seg(g_hbm, src_hbm, dst_hbm, out_hbm, src_v, dst_v, *bufs_sems):
        rows = bufs_sems[:NBUF]
        acc = bufs_sems[NBUF]
        gsem = bufs_sems[NBUF + 1:2 * NBUF + 1]
        ssem = bufs_sems[2 * NBUF + 1:3 * NBUF + 1]
        c = lax.axis_index("c")
        s = lax.axis_index("s")
        w = c * NTILES + s
        nz = Df // 16

        def zrow(i, carry):
            for q in range(nz):
                rows[0][i, pl.ds(q * 16, 16)] = jnp.zeros((16,), jnp.float32)
            return carry
        lax.fori_loop(0, K, zrow, 0)
        base = s * RPT
        for off, sz in _ZBLK:
            pltpu.sync_copy(rows[0].at[pl.ds(0, sz)],
                            acc.at[pl.ds(base + off, sz)])
        plsc.subcore_barrier()

        for half in range(NHALF):
            cb = w * C + half * C2
            nreal = jnp.clip(ch_proc - cb, 0, C2)
            pltpu.sync_copy(src_hbm.at[pl.ds(cb, C2)], src_v)
            pltpu.sync_copy(dst_hbm.at[pl.ds(cb, C2)], dst_v)

            for b in range(NBUF):
                @pl.when(b < nreal)
                def _(b=b):
                    pltpu.async_copy(g_hbm.at[src_v.at[b]], rows[b], gsem[b])

            def group(gi, carry):
                for b in range(NBUF):
                    j = gi * NBUF + b
                    pltpu.make_async_copy(
                        g_hbm.at[src_v.at[j]], rows[b], gsem[b]).wait()
                    pltpu.async_copy(
                        rows[b], acc.at[dst_v.at[j]], ssem[b], add=True)
                    jn = j + NBUF

                    @pl.when(jn < nreal)
                    def _(b=b, j=j, jn=jn):
                        pltpu.make_async_copy(
                            rows[b], acc.at[dst_v.at[j]], ssem[b]).wait()
                        pltpu.async_copy(
                            g_hbm.at[src_v.at[jn]], rows[b], gsem[b])
                return carry
            lax.fori_loop(0, nreal // NBUF, group, 0)
            for b in range(NBUF):
                @pl.when(b < nreal)
                def _(b=b):
                    pltpu.make_async_copy(
                        rows[b], acc.at[dst_v.at[0]], ssem[b]).wait()
        plsc.subcore_barrier()
        for off, sz in _ZBLK:
            pltpu.sync_copy(acc.at[pl.ds(base + off, sz)],
                            out_hbm.at[c, pl.ds(base + off, sz)])

    return seg


def _dinv_of(deg_ref):
    return lax.rsqrt(deg_ref[0, :, 0:1] + deg_ref[1, :, 0:1] + 1.0)


def _mm1_body(x_ref, w_ref, deg_ref, o_ref):
    o_ref[:] = jnp.dot(x_ref[:], w_ref[:],
                       preferred_element_type=jnp.float32,
                       precision=lax.Precision.HIGHEST) * _dinv_of(deg_ref)


def _mid_body(s_ref, g_ref, deg_ref, b_ref, w_ref, o_ref):
    dinv = _dinv_of(deg_ref)
    h = (s_ref[0] + s_ref[1] + g_ref[:]) * dinv + b_ref[:]
    h = jnp.maximum(h, 0.0)
    o_ref[:] = jnp.dot(h, w_ref[:],
                       preferred_element_type=jnp.float32,
                       precision=lax.Precision.HIGHEST) * dinv


def _fin_body(s_ref, g_ref, deg_ref, b_ref, o_ref):
    dinv = _dinv_of(deg_ref)
    h = (s_ref[0] + s_ref[1] + g_ref[:]) * dinv + b_ref[:]
    z = h[:, :NCLS]
    m = jnp.max(z, axis=1, keepdims=True)
    lse = jnp.log(jnp.sum(jnp.exp(z - m), axis=1, keepdims=True)) + m
    o_ref[:] = z - lse


def _mm1(x, w1, deg):
    return pl.pallas_call(
        _mm1_body,
        grid=(N // BR,),
        in_specs=[
            pl.BlockSpec((BR, D), lambda i: (i, 0)),
            pl.BlockSpec((D, D), lambda i: (0, 0)),
            pl.BlockSpec((NCORES, BR, 16), lambda i: (0, i, 0)),
        ],
        out_specs=pl.BlockSpec((BR, D), lambda i: (i, 0)),
        out_shape=jax.ShapeDtypeStruct((N, D), jnp.float32),
    )(x, w1, deg)


def _mid(s, g, deg, b, w):
    din, dout = w.shape
    return pl.pallas_call(
        _mid_body,
        grid=(N // BR,),
        in_specs=[
            pl.BlockSpec((NCORES, BR, din), lambda i: (0, i, 0)),
            pl.BlockSpec((BR, din), lambda i: (i, 0)),
            pl.BlockSpec((NCORES, BR, 16), lambda i: (0, i, 0)),
            pl.BlockSpec((1, din), lambda i: (0, 0)),
            pl.BlockSpec((din, dout), lambda i: (0, 0)),
        ],
        out_specs=pl.BlockSpec((BR, dout), lambda i: (i, 0)),
        out_shape=jax.ShapeDtypeStruct((N, dout), jnp.float32),
    )(s, g, deg, b.reshape(1, din), w)


def _fin(s, g, deg, b):
    return pl.pallas_call(
        _fin_body,
        grid=(N // BR,),
        in_specs=[
            pl.BlockSpec((NCORES, BR, NCLS_PAD), lambda i: (0, i, 0)),
            pl.BlockSpec((BR, NCLS_PAD), lambda i: (i, 0)),
            pl.BlockSpec((NCORES, BR, 16), lambda i: (0, i, 0)),
            pl.BlockSpec((1, NCLS_PAD), lambda i: (0, 0)),
        ],
        out_specs=pl.BlockSpec((BR, NCLS), lambda i: (i, 0)),
        out_shape=jax.ShapeDtypeStruct((N, NCLS), jnp.float32),
    )(s, g, deg, b.reshape(1, NCLS_PAD))


def kernel(x, edge_index, W1, b1, W2, b2, W3, b3, W4, b4, W5, b5):
    src = edge_index[0].astype(jnp.int32)
    dst = edge_index[1].astype(jnp.int32)
    e = src.shape[0]
    ch = -(-e // K)
    c_per_w = ((ch + NW - 1) // NW + 7) // 8 * 8
    ch_pad = c_per_w * NW
    ep = ch_pad * K
    ch_proc = -(-ch // 4) * 4
    pad_dst = N + 1 + (jnp.arange(ep - e, dtype=jnp.int32) % (N_PAD - N - 1))
    src_p = jnp.concatenate(
        [src, jnp.zeros((ep - e,), jnp.int32)]).reshape(ch_pad, K)
    dst_p = jnp.concatenate([dst, pad_dst]).reshape(ch_pad, K)

    w5p = jnp.pad(W5, ((0, 0), (0, NCLS_PAD - NCLS)))
    b5p = jnp.pad(b5, (0, NCLS_PAD - NCLS))

    deg = _make_deg(c_per_w, ch_proc)(dst_p)
    seg128 = _make_seg(D, c_per_w, ch_proc)

    g = _mm1(x, W1, deg)
    for b, w in ((b1, W2), (b2, W3), (b3, W4), (b4, w5p)):
        s = seg128(g, src_p, dst_p)
        g = _mid(s, g, deg, b, w)
    s = seg128(g, src_p, dst_p)
    return _fin(s, g, deg, b5p)
